# Initial kernel scaffold; baseline (speedup 1.0000x reference)
import jax
import jax.numpy as jnp
from jax.experimental import pallas as pl


def kernel(center_tiles, context_tiles, target_table, context_table):
    raise NotImplementedError("write your pallas kernel here")


# speedup vs baseline: 1.0716x; 1.0716x over previous
import functools

import jax
import jax.numpy as jnp
from jax import lax
from jax.experimental import pallas as pl
from jax.experimental.pallas import tpu as pltpu
from jax.experimental.pallas import tpu_sc as plsc

_VOCAB = 1000000
_DIM = 32
_B = 16384
_C = 20

_NC = 2
_NS = 16
_NW = _NC * _NS
_BPW = _B // _NW
_CHUNK = 64
_NCHUNK = _BPW // _CHUNK
_CTX_PER_CHUNK = _CHUNK * _C
_IDXW = 128
_CIDX_ROWS = _BPW // _IDXW
_XIDX_ROWS = (_BPW * _C) // _IDXW
_XROWS_PER_CHUNK = _CTX_PER_CHUNK // _IDXW

_LN2 = 0.6931471805599453


def _vlog(x):
  bits = plsc.bitcast(x, jnp.int32)
  e = (bits >> 23) - 127
  m = plsc.bitcast((bits & 0x7FFFFF) | 0x3F800000, jnp.float32)
  t = (m - 1.0) / (m + 1.0)
  t2 = t * t
  p = jnp.float32(1.0 / 11.0)
  p = p * t2 + jnp.float32(1.0 / 9.0)
  p = p * t2 + jnp.float32(1.0 / 7.0)
  p = p * t2 + jnp.float32(1.0 / 5.0)
  p = p * t2 + jnp.float32(1.0 / 3.0)
  p = p * t2 + jnp.float32(1.0)
  return e.astype(jnp.float32) * jnp.float32(_LN2) + 2.0 * t * p


def _body(cidx_hbm, xidx_hbm, target_hbm, context_hbm, out_hbm,
          cidx_v, xidx_v, crows, xbuf0, xbuf1, svals, ovec,
          sem_c, sem_x0, sem_x1):
  wid = lax.axis_index("s") * _NC + lax.axis_index("c")

  pltpu.sync_copy(cidx_hbm.at[pl.ds(wid * _CIDX_ROWS, _CIDX_ROWS)], cidx_v)
  pltpu.sync_copy(xidx_hbm.at[pl.ds(wid * _XIDX_ROWS, _XIDX_ROWS)], xidx_v)

  for r in range(_CIDX_ROWS):
    pltpu.make_async_copy(
        target_hbm.at[cidx_v.at[r]],
        crows.at[pl.ds(r * _IDXW, _IDXW)], sem_c).start()

  xbufs = (xbuf0, xbuf1)
  sems = (sem_x0, sem_x1)

  def start_ctx(j):
    buf = xbufs[j % 2]
    for k in range(_XROWS_PER_CHUNK):
      pltpu.make_async_copy(
          context_hbm.at[xidx_v.at[j * _XROWS_PER_CHUNK + k]],
          buf.at[pl.ds(k * _IDXW, _IDXW)], sems[j % 2]).start()

  def wait_ctx(j):
    buf = xbufs[j % 2]
    for k in range(_XROWS_PER_CHUNK):
      pltpu.make_async_copy(
          context_hbm.at[xidx_v.at[j * _XROWS_PER_CHUNK + k]],
          buf.at[pl.ds(k * _IDXW, _IDXW)], sems[j % 2]).wait()

  start_ctx(0)
  for r in range(_CIDX_ROWS):
    pltpu.make_async_copy(
        target_hbm.at[cidx_v.at[r]],
        crows.at[pl.ds(r * _IDXW, _IDXW)], sem_c).wait()

  lane = lax.iota(jnp.int32, 16)
  tail_mask = lane < (_C - 16)
  zeros = jnp.zeros((16,), jnp.float32)

  t_acc = zeros
  for j in range(_NCHUNK):
    if j + 1 < _NCHUNK:
      start_ctx(j + 1)
    wait_ctx(j)
    xb = xbufs[j % 2]

    def row_body(r, t_acc, j=j, xb=xb):
      row = j * _CHUNK + r
      c0 = crows[row, pl.ds(0, 16)]
      c1 = crows[row, pl.ds(16, 16)]
      l1 = zeros
      l2 = zeros
      for c in range(_C):
        x0 = xb[r * _C + c, pl.ds(0, 16)]
        x1 = xb[r * _C + c, pl.ds(16, 16)]
        s = jnp.sum(c0 * x0 + c1 * x1)
        if c < 16:
          l1 = jnp.where(lane == c, s, l1)
        else:
          l2 = jnp.where(lane == c - 16, s, l2)
      e = jnp.exp(l1) + jnp.where(tail_mask, jnp.exp(l2), 0.0)
      plsc.store_scatter(svals, [jnp.broadcast_to(row, (16,))],
                         jnp.broadcast_to(jnp.sum(e), (16,)),
                         mask=lane == 0)
      return t_acc + l1 + jnp.where(tail_mask, l2, zeros)

    t_acc = lax.fori_loop(0, _CHUNK, row_body, t_acc)

  def log_body(g, acc):
    return acc + _vlog(svals[pl.ds(g * 16, 16)])

  lse_acc = lax.fori_loop(0, _BPW // 16, log_body, zeros)

  p_lse = jnp.sum(lse_acc)
  p_t = jnp.sum(t_acc)
  ovec[...] = jnp.where(lane == 0, p_lse, jnp.where(lane == 1, p_t, 0.0))
  pltpu.sync_copy(ovec, out_hbm.at[wid])


@jax.jit
def _block2vec_partials(cidx, xidx, target_table, context_table):
  mesh = plsc.VectorSubcoreMesh(core_axis_name="c", subcore_axis_name="s")
  f = pl.kernel(
      _body,
      out_type=jax.ShapeDtypeStruct((_NW, 16), jnp.float32),
      mesh=mesh,
      compiler_params=pltpu.CompilerParams(
          needs_layout_passes=False, use_tc_tiling_on_sc=False),
      scratch_types=[
          pltpu.VMEM((_CIDX_ROWS, _IDXW), jnp.int32),
          pltpu.VMEM((_XIDX_ROWS, _IDXW), jnp.int32),
          pltpu.VMEM((_BPW, _DIM), jnp.float32),
          pltpu.VMEM((_CTX_PER_CHUNK, _DIM), jnp.float32),
          pltpu.VMEM((_CTX_PER_CHUNK, _DIM), jnp.float32),
          pltpu.VMEM((_BPW,), jnp.float32),
          pltpu.VMEM((16,), jnp.float32),
          pltpu.SemaphoreType.DMA,
          pltpu.SemaphoreType.DMA,
          pltpu.SemaphoreType.DMA,
      ],
  )
  return f(cidx, xidx, target_table, context_table)


def kernel(center_tiles, context_tiles, target_table, context_table):
  cidx = center_tiles.astype(jnp.int32).reshape(_B // _IDXW, _IDXW)
  xidx = context_tiles.astype(jnp.int32).reshape((_B * _C) // _IDXW, _IDXW)
  partials = _block2vec_partials(cidx, xidx, target_table, context_table)
  sum_lse = jnp.sum(partials[:, 0])
  sum_logits = jnp.sum(partials[:, 1])
  return sum_lse / _B - sum_logits / (_B * _C)
